# baseline (speedup 1.0000x reference)
import functools
import math

import numpy as np
import jax
import jax.numpy as jnp
from jax import lax
from jax.experimental import pallas as pl
from jax.experimental.pallas import tpu as pltpu
from jax.experimental.pallas import tpu_sc as plsc

_TIMESTEPS = 1000


def _make_table():
    T = _TIMESTEPS
    ts = np.linspace(0.0, 1.0, T + 1, dtype=np.float32)
    s = np.float32(0.008)
    abar = np.cos((ts + s) / (1 + s) * np.float32(math.pi / 2)) ** 2
    abar = abar / abar[0]
    alphas_tail = abar[1:T] / abar[0 : T - 1]
    betas_tail = np.clip(1.0 - alphas_tail, 1e-08, 0.999).astype(np.float32)
    betas = np.concatenate([np.zeros((1,), dtype=np.float32), betas_tail])
    alpha_bars = np.cumprod((1.0 - betas).astype(np.float32))
    tab = np.zeros((T, 128), dtype=np.float32)
    tab[:, 0] = np.sqrt(alpha_bars)
    tab[:, 1] = np.sqrt(1.0 - alpha_bars)
    return tab


_TABLE = _make_table()


def _sc_gather(t_idx, tab):
    n = t_idx.shape[0]
    info = plsc.get_sparse_core_info()
    nc, ns, lanes = info.num_cores, info.num_subcores, info.num_lanes
    nw = nc * ns
    per_w = n // nw

    @functools.partial(
        pl.kernel,
        mesh=plsc.VectorSubcoreMesh(core_axis_name="c", subcore_axis_name="s"),
        out_type=jax.ShapeDtypeStruct((n, 128), jnp.float32),
        scratch_types=[
            pltpu.VMEM((per_w,), jnp.int32),
            pltpu.VMEM((per_w, 128), jnp.float32),
            pltpu.SemaphoreType.DMA,
        ],
    )
    def gather_kernel(t_hbm, tab_hbm, out_hbm, idx_v, rows_v, sem):
        wid = lax.axis_index("s") * nc + lax.axis_index("c")
        base = wid * per_w
        pltpu.sync_copy(t_hbm.at[pl.ds(base, per_w)], idx_v)
        for i in range(per_w // lanes):
            sl = pl.ds(i * lanes, lanes)
            idx_v[sl] = jnp.minimum(jnp.maximum(idx_v[sl], 0), _TIMESTEPS - 1)
        pltpu.async_copy(tab_hbm.at[idx_v], rows_v, sem).wait()
        pltpu.sync_copy(rows_v, out_hbm.at[pl.ds(base, per_w)])

    return gather_kernel(t_idx, tab)


def _sc_copy(nv):
    d, b = nv.shape
    info = plsc.get_sparse_core_info()
    nc, ns = info.num_cores, info.num_subcores
    nw = nc * ns
    per_w = d // nw

    @functools.partial(
        pl.kernel,
        mesh=plsc.VectorSubcoreMesh(core_axis_name="c", subcore_axis_name="s"),
        out_type=jax.ShapeDtypeStruct((d, b), jnp.float32),
    )
    def copy_kernel(src_hbm, dst_hbm):
        wid = lax.axis_index("s") * nc + lax.axis_index("c")
        base = wid * per_w
        sl = pl.ds(base, per_w)
        pltpu.sync_copy(src_hbm.at[sl], dst_hbm.at[sl])

    return copy_kernel(nv)


def _fma_body(sa_ref, sb_ref, x_ref, n_ref, o_ref):
    o_ref[...] = sa_ref[...] * x_ref[...] + sb_ref[...] * n_ref[...]


def _tc_fma(sa, sb, xv, nv):
    d, b = xv.shape
    rr = 1024
    grid = (d // rr,)
    spec_s = pl.BlockSpec((1, b), lambda i: (0, 0))
    spec_x = pl.BlockSpec((rr, b), lambda i: (i, 0))
    return pl.pallas_call(
        _fma_body,
        grid=grid,
        in_specs=[spec_s, spec_s, spec_x, spec_x],
        out_specs=spec_x,
        out_shape=jax.ShapeDtypeStruct((d, b), jnp.float32),
        compiler_params=pltpu.CompilerParams(
            dimension_semantics=("parallel",),
            vmem_limit_bytes=100 * 1024 * 1024,
        ),
    )(sa, sb, xv, nv)


def kernel(x0, t, noise):
    b = x0.shape[0]
    tail = x0.shape[1:]
    d = int(np.prod(tail))
    rows = _sc_gather(t.astype(jnp.int32), jnp.asarray(_TABLE))
    sa = rows[:, 0].reshape(1, b)
    sb = rows[:, 1].reshape(1, b)
    xv = jnp.transpose(x0, (1, 2, 3, 0)).reshape(d, b)
    nv = jnp.transpose(noise, (1, 2, 3, 0)).reshape(d, b)
    xt = _tc_fma(sa, sb, xv, nv)
    nout = _sc_copy(nv)
    xt = jnp.transpose(xt.reshape(tail + (b,)), (3, 0, 1, 2))
    nout = jnp.transpose(nout.reshape(tail + (b,)), (3, 0, 1, 2))
    return xt, nout


# speedup vs baseline: 19.0233x; 19.0233x over previous
import functools
import math

import numpy as np
import jax
import jax.numpy as jnp
from jax import lax
from jax.experimental import pallas as pl
from jax.experimental.pallas import tpu as pltpu
from jax.experimental.pallas import tpu_sc as plsc

_TIMESTEPS = 1000


def _make_table():
    T = _TIMESTEPS
    ts = np.linspace(0.0, 1.0, T + 1, dtype=np.float32)
    s = np.float32(0.008)
    abar = np.cos((ts + s) / (1 + s) * np.float32(math.pi / 2)) ** 2
    abar = abar / abar[0]
    alphas_tail = abar[1:T] / abar[0 : T - 1]
    betas_tail = np.clip(1.0 - alphas_tail, 1e-08, 0.999).astype(np.float32)
    betas = np.concatenate([np.zeros((1,), dtype=np.float32), betas_tail])
    alpha_bars = np.cumprod((1.0 - betas).astype(np.float32))
    tab = np.zeros((T, 128), dtype=np.float32)
    tab[:, 0] = np.sqrt(alpha_bars)
    tab[:, 1] = np.sqrt(1.0 - alpha_bars)
    return tab


_TABLE = _make_table()


def _sc_gather(t_idx, tab):
    n = t_idx.shape[0]
    info = plsc.get_sparse_core_info()
    nc, ns, lanes = info.num_cores, info.num_subcores, info.num_lanes
    nw = nc * ns
    per_w = n // nw

    @functools.partial(
        pl.kernel,
        mesh=plsc.VectorSubcoreMesh(core_axis_name="c", subcore_axis_name="s"),
        out_type=jax.ShapeDtypeStruct((n, 128), jnp.float32),
        scratch_types=[
            pltpu.VMEM((per_w,), jnp.int32),
            pltpu.VMEM((per_w, 128), jnp.float32),
            pltpu.SemaphoreType.DMA,
        ],
    )
    def gather_kernel(t_hbm, tab_hbm, out_hbm, idx_v, rows_v, sem):
        wid = lax.axis_index("s") * nc + lax.axis_index("c")
        base = wid * per_w
        pltpu.sync_copy(t_hbm.at[pl.ds(base, per_w)], idx_v)
        for i in range(per_w // lanes):
            sl = pl.ds(i * lanes, lanes)
            idx_v[sl] = jnp.minimum(jnp.maximum(idx_v[sl], 0), _TIMESTEPS - 1)
        pltpu.async_copy(tab_hbm.at[idx_v], rows_v, sem).wait()
        pltpu.sync_copy(rows_v, out_hbm.at[pl.ds(base, per_w)])

    return gather_kernel(t_idx, tab)


def _fma_body(sa_ref, sb_ref, x_ref, n_ref, o_ref, no_ref):
    nz = n_ref[...]
    o_ref[...] = sa_ref[...] * x_ref[...] + sb_ref[...] * nz
    no_ref[...] = nz


def _tc_fma(sa, sb, xv, nv):
    d, b = xv.shape
    rr = 1024
    grid = (d // rr,)
    spec_s = pl.BlockSpec((1, b), lambda i: (0, 0))
    spec_x = pl.BlockSpec((rr, b), lambda i: (i, 0))
    return pl.pallas_call(
        _fma_body,
        grid=grid,
        in_specs=[spec_s, spec_s, spec_x, spec_x],
        out_specs=[spec_x, spec_x],
        out_shape=[
            jax.ShapeDtypeStruct((d, b), jnp.float32),
            jax.ShapeDtypeStruct((d, b), jnp.float32),
        ],
        compiler_params=pltpu.CompilerParams(
            dimension_semantics=("parallel",),
            vmem_limit_bytes=100 * 1024 * 1024,
        ),
    )(sa, sb, xv, nv)


def kernel(x0, t, noise):
    b = x0.shape[0]
    tail = x0.shape[1:]
    d = int(np.prod(tail))
    rows = _sc_gather(t.astype(jnp.int32), jnp.asarray(_TABLE))
    sa = rows[:, 0].reshape(1, b)
    sb = rows[:, 1].reshape(1, b)
    xv = jnp.transpose(x0, (1, 2, 3, 0)).reshape(d, b)
    nv = jnp.transpose(noise, (1, 2, 3, 0)).reshape(d, b)
    xt, nout = _tc_fma(sa, sb, xv, nv)
    xt = jnp.transpose(xt.reshape(tail + (b,)), (3, 0, 1, 2))
    nout = jnp.transpose(nout.reshape(tail + (b,)), (3, 0, 1, 2))
    return xt, nout


# speedup vs baseline: 19.2219x; 1.0104x over previous
import functools
import math

import numpy as np
import jax
import jax.numpy as jnp
from jax import lax
from jax.experimental import pallas as pl
from jax.experimental.pallas import tpu as pltpu
from jax.experimental.pallas import tpu_sc as plsc

_TIMESTEPS = 1000


def _make_table():
    T = _TIMESTEPS
    ts = np.linspace(0.0, 1.0, T + 1, dtype=np.float32)
    s = np.float32(0.008)
    abar = np.cos((ts + s) / (1 + s) * np.float32(math.pi / 2)) ** 2
    abar = abar / abar[0]
    alphas_tail = abar[1:T] / abar[0 : T - 1]
    betas_tail = np.clip(1.0 - alphas_tail, 1e-08, 0.999).astype(np.float32)
    betas = np.concatenate([np.zeros((1,), dtype=np.float32), betas_tail])
    alpha_bars = np.cumprod((1.0 - betas).astype(np.float32))
    tab = np.zeros((T, 128), dtype=np.float32)
    tab[:, 0] = np.sqrt(alpha_bars)
    tab[:, 1] = np.sqrt(1.0 - alpha_bars)
    return tab


_TABLE = _make_table()


def _sc_gather(t_idx, tab):
    n = t_idx.shape[0]
    info = plsc.get_sparse_core_info()
    nc, ns, lanes = info.num_cores, info.num_subcores, info.num_lanes
    nw = nc * ns
    per_w = n // nw

    @functools.partial(
        pl.kernel,
        mesh=plsc.VectorSubcoreMesh(core_axis_name="c", subcore_axis_name="s"),
        out_type=jax.ShapeDtypeStruct((n, 128), jnp.float32),
        scratch_types=[
            pltpu.VMEM((per_w,), jnp.int32),
            pltpu.VMEM((per_w, 128), jnp.float32),
            pltpu.SemaphoreType.DMA,
        ],
    )
    def gather_kernel(t_hbm, tab_hbm, out_hbm, idx_v, rows_v, sem):
        wid = lax.axis_index("s") * nc + lax.axis_index("c")
        base = wid * per_w
        pltpu.sync_copy(t_hbm.at[pl.ds(base, per_w)], idx_v)
        for i in range(per_w // lanes):
            sl = pl.ds(i * lanes, lanes)
            idx_v[sl] = jnp.minimum(jnp.maximum(idx_v[sl], 0), _TIMESTEPS - 1)
        pltpu.async_copy(tab_hbm.at[idx_v], rows_v, sem).wait()
        pltpu.sync_copy(rows_v, out_hbm.at[pl.ds(base, per_w)])

    return gather_kernel(t_idx, tab)


def _fma_body(rows_ref, x_ref, n_ref, o_ref, no_ref, s_scr):
    @pl.when(pl.program_id(0) == 0)
    def _init():
        s_scr[...] = jnp.swapaxes(rows_ref[:, 0:2], 0, 1)

    nz = n_ref[...]
    o_ref[...] = s_scr[0:1, :] * x_ref[...] + s_scr[1:2, :] * nz
    no_ref[...] = nz


def _tc_fma(rows, xv, nv):
    d, b = xv.shape
    rr = 1024
    grid = (d // rr,)
    spec_r = pl.BlockSpec((b, 128), lambda i: (0, 0))
    spec_x = pl.BlockSpec((rr, b), lambda i: (i, 0))
    return pl.pallas_call(
        _fma_body,
        grid=grid,
        in_specs=[spec_r, spec_x, spec_x],
        out_specs=[spec_x, spec_x],
        out_shape=[
            jax.ShapeDtypeStruct((d, b), jnp.float32),
            jax.ShapeDtypeStruct((d, b), jnp.float32),
        ],
        scratch_shapes=[pltpu.VMEM((2, b), jnp.float32)],
        compiler_params=pltpu.CompilerParams(
            dimension_semantics=("arbitrary",),
            vmem_limit_bytes=100 * 1024 * 1024,
        ),
    )(rows, xv, nv)


def kernel(x0, t, noise):
    b = x0.shape[0]
    tail = x0.shape[1:]
    d = int(np.prod(tail))
    rows = _sc_gather(t.astype(jnp.int32), jnp.asarray(_TABLE))
    xv = jnp.transpose(x0, (1, 2, 3, 0)).reshape(d, b)
    nv = jnp.transpose(noise, (1, 2, 3, 0)).reshape(d, b)
    xt, nout = _tc_fma(rows, xv, nv)
    xt = jnp.transpose(xt.reshape(tail + (b,)), (3, 0, 1, 2))
    nout = jnp.transpose(nout.reshape(tail + (b,)), (3, 0, 1, 2))
    return xt, nout
